# baseline (speedup 1.0000x reference)
import jax
import jax.numpy as jnp
from jax import lax
from jax.experimental import pallas as pl
from jax.experimental.pallas import tpu as pltpu
from jax.experimental.pallas import tpu_sc as plsc

EMBED_DIM = 64
NC = 2
NS = 16
NW = NC * NS
CB = 512


def _gather_body(x_hbm, table_hbm, out_hbm, idx_v, rows_v, sem):
    n = x_hbm.shape[0]
    b_per_w = n // NW
    nchunks = b_per_w // CB
    wid = lax.axis_index("s") * NC + lax.axis_index("c")
    base = wid * b_per_w

    @pl.loop(0, nchunks)
    def _chunk(c):
        off = base + c * CB
        pltpu.sync_copy(x_hbm.at[pl.ds(off, CB)], idx_v)
        pltpu.async_copy(table_hbm.at[idx_v], rows_v, sem).wait()
        pltpu.sync_copy(rows_v, out_hbm.at[pl.ds(off, CB)])


def kernel(x, table):
    b, h = x.shape
    n = b * h
    flat = x.reshape(n)
    mesh = plsc.VectorSubcoreMesh(core_axis_name="c", subcore_axis_name="s")
    out = pl.kernel(
        _gather_body,
        out_type=jax.ShapeDtypeStruct((n, EMBED_DIM), jnp.float32),
        mesh=mesh,
        scratch_types=[
            pltpu.VMEM((CB,), jnp.int32),
            pltpu.VMEM((CB, EMBED_DIM), jnp.float32),
            pltpu.SemaphoreType.DMA,
        ],
        compiler_params=pltpu.CompilerParams(use_tc_tiling_on_sc=False),
    )(flat, table)
    return out.reshape(b, h, EMBED_DIM)


# speedup vs baseline: 1.0459x; 1.0459x over previous
import jax
import jax.numpy as jnp
from jax import lax
from jax.experimental import pallas as pl
from jax.experimental.pallas import tpu as pltpu
from jax.experimental.pallas import tpu_sc as plsc

EMBED_DIM = 64
NC = 2
NS = 16
NW = NC * NS
CB = 256
NBUF = 4
PIPE_D = 2


def _gather_body(x_hbm, table_hbm, out_hbm, idx_all, rows, gsems, ssems):
    n = x_hbm.shape[0]
    b_per_w = n // NW
    nchunks = b_per_w // CB
    nlaps = nchunks // NBUF
    wid = lax.axis_index("s") * NC + lax.axis_index("c")
    base = wid * b_per_w

    pltpu.sync_copy(x_hbm.at[pl.ds(base, b_per_w)], idx_all)

    def issue_gather(c, b):
        pltpu.async_copy(
            table_hbm.at[idx_all.at[pl.ds(c * CB, CB)]], rows[b], gsems[b])

    def wait_gather(b):
        pltpu.make_async_copy(
            table_hbm.at[idx_all.at[pl.ds(0, CB)]], rows[b], gsems[b]).wait()

    def issue_store(c, b):
        pltpu.async_copy(rows[b], out_hbm.at[pl.ds(base + c * CB, CB)], ssems[b])

    def wait_store(b):
        pltpu.make_async_copy(
            rows[b], out_hbm.at[pl.ds(base, CB)], ssems[b]).wait()

    for c in range(PIPE_D):
        issue_gather(c, c % NBUF)
    for c in range(PIPE_D, NBUF):
        issue_gather(c, c % NBUF)
        wait_gather((c - PIPE_D) % NBUF)
        issue_store(c - PIPE_D, (c - PIPE_D) % NBUF)

    @pl.loop(1, nlaps)
    def _lap(g):
        for b in range(NBUF):
            c = g * NBUF + b
            wait_store(b)
            issue_gather(c, b)
            wait_gather((b - PIPE_D) % NBUF)
            issue_store(c - PIPE_D, (b - PIPE_D) % NBUF)

    for k in range(PIPE_D):
        b = (NBUF - PIPE_D + k) % NBUF
        wait_gather(b)
        issue_store(nchunks - PIPE_D + k, b)
    for b in range(NBUF):
        wait_store(b)


def kernel(x, table):
    b, h = x.shape
    n = b * h
    flat = x.reshape(n)
    mesh = plsc.VectorSubcoreMesh(core_axis_name="c", subcore_axis_name="s")
    out = pl.kernel(
        _gather_body,
        out_type=jax.ShapeDtypeStruct((n, EMBED_DIM), jnp.float32),
        mesh=mesh,
        scratch_types=[
            pltpu.VMEM((n // NW,), jnp.int32),
            [pltpu.VMEM((CB, EMBED_DIM), jnp.float32) for _ in range(NBUF)],
            [pltpu.SemaphoreType.DMA for _ in range(NBUF)],
            [pltpu.SemaphoreType.DMA for _ in range(NBUF)],
        ],
        compiler_params=pltpu.CompilerParams(use_tc_tiling_on_sc=False),
    )(flat, table)
    return out.reshape(b, h, EMBED_DIM)
